# Initial kernel scaffold; baseline (speedup 1.0000x reference)
import jax
import jax.numpy as jnp
from jax.experimental import pallas as pl


def kernel(img_tokens, W1, b1, W2, b2, vstart, vend, table, inst_ids, ans_ids, cu_inst, cu_ans):
    raise NotImplementedError("write your pallas kernel here")


# speedup vs baseline: 1.0748x; 1.0748x over previous
import functools

import jax
import jax.numpy as jnp
from jax import lax
from jax.experimental import pallas as pl
from jax.experimental.pallas import tpu as pltpu
from jax.experimental.pallas import tpu_sc as plsc

B = 4
N_IMG = 576
IN_CH = 1024
D = 2048
MAXLEN = 2048
N_IMG_BLK = N_IMG + 2

NC = 2
NS = 16
NW = NC * NS
LANES = 16

ROWS = B * MAXLEN
ROWS_PER_W = ROWS // NW
CHUNK = 16
N_CHUNKS = ROWS_PER_W // CHUNK


def _round_up(n, m):
    return (n + m - 1) // m * m


def _sc_gather_body(table, inst, ans, cu, img, out,
                    rows0, rows1, inst_v, ans_v, cu_v, sem0, sem1):
    cid = lax.axis_index("c")
    sid = lax.axis_index("s")
    b = cid * 2 + sid // 8
    sub8 = sid % 8
    po = sub8 * ROWS_PER_W
    base = b * MAXLEN + po

    pltpu.sync_copy(inst, inst_v)
    pltpu.sync_copy(ans, ans_v)
    pltpu.sync_copy(cu, cu_v)

    bvec = jnp.full((LANES,), b, dtype=jnp.int32)
    cu_i_b = plsc.load_gather(cu_v, [bvec])
    cu_i_b1 = plsc.load_gather(cu_v, [bvec + 1])
    cu_a_b = plsc.load_gather(cu_v, [bvec + 8])
    cu_a_b1 = plsc.load_gather(cu_v, [bvec + 9])
    li = cu_i_b1 - cu_i_b
    la = cu_a_b1 - cu_a_b
    ni = inst_v.shape[0]
    na = ans_v.shape[0]
    lane = lax.iota(jnp.int32, LANES)

    def tok_for(c):
        p = jnp.full((LANES,), po + c * LANES, dtype=jnp.int32) + lane
        inst_addr = jnp.clip(cu_i_b + p, 0, ni - 1)
        aoff = p - li - N_IMG_BLK
        ans_addr = jnp.clip(cu_a_b + aoff, 0, na - 1)
        ti = plsc.load_gather(inst_v, [inst_addr])
        ta = plsc.load_gather(ans_v, [ans_addr])
        is_inst = p < li
        is_ans = jnp.logical_and(aoff >= 0, aoff < la)
        return jnp.where(is_inst, ti, jnp.where(is_ans, ta, 0))

    bufs = (rows0, rows1)
    sems = (sem0, sem1)
    handles = [None, None]
    for c in range(N_CHUNKS):
        s = c % 2
        if handles[s] is not None:
            handles[s].wait()
            pltpu.sync_copy(bufs[s],
                            out.at[pl.ds(base + (c - 2) * CHUNK, CHUNK)])
        tok = tok_for(c)
        handles[s] = pltpu.async_copy(table.at[tok], bufs[s], sems[s])
    for c in range(N_CHUNKS - 2, N_CHUNKS):
        s = c % 2
        handles[s].wait()
        pltpu.sync_copy(bufs[s], out.at[pl.ds(base + c * CHUNK, CHUNK)])

    plsc.subcore_barrier()

    n_img_chunks = (N_IMG_BLK + CHUNK - 1) // CHUNK
    for i in range((n_img_chunks + 7) // 8):
        c = sub8 + i * 8

        @pl.when(c < n_img_chunks)
        def _():
            j = jnp.minimum(c * CHUNK, N_IMG_BLK - CHUNK)
            src_idx = jnp.full((LANES,), b * N_IMG_BLK, jnp.int32) + j + lane
            dst_idx = li + (b * MAXLEN + j) + lane
            pltpu.async_copy(img.at[src_idx], rows0, sem0).wait()
            pltpu.async_copy(rows0, out.at[dst_idx], sem0).wait()


def _sc_gather(table, inst_p, ans_p, cu16, img_flat):
    ni = inst_p.shape[0]
    na = ans_p.shape[0]
    mesh = plsc.VectorSubcoreMesh(core_axis_name="c", subcore_axis_name="s")
    return pl.kernel(
        _sc_gather_body,
        out_type=jax.ShapeDtypeStruct((ROWS, D), jnp.float32),
        mesh=mesh,
        compiler_params=pltpu.CompilerParams(needs_layout_passes=False),
        scratch_types=[
            pltpu.VMEM((CHUNK, D), jnp.float32),
            pltpu.VMEM((CHUNK, D), jnp.float32),
            pltpu.VMEM((ni,), jnp.int32),
            pltpu.VMEM((na,), jnp.int32),
            pltpu.VMEM((16,), jnp.int32),
            pltpu.SemaphoreType.DMA,
            pltpu.SemaphoreType.DMA,
        ],
    )(table, inst_p, ans_p, cu16, img_flat)


def _mlp_body(x_ref, w1_ref, b1_ref, w2_ref, b2_ref, vs_ref, ve_ref, out_ref):
    x = x_ref[0]
    h = jnp.dot(x, w1_ref[...], preferred_element_type=jnp.float32)
    h = jax.nn.gelu(h + b1_ref[...])
    p = jnp.dot(h, w2_ref[...], preferred_element_type=jnp.float32)
    p = p + b2_ref[...]
    out_ref[0] = jnp.concatenate([vs_ref[...], p, ve_ref[...]], axis=0)


def _mlp(img_tokens, W1, b1, W2, b2, vstart, vend):
    return pl.pallas_call(
        _mlp_body,
        grid=(B,),
        in_specs=[
            pl.BlockSpec((1, N_IMG, IN_CH), lambda i: (i, 0, 0)),
            pl.BlockSpec((IN_CH, D), lambda i: (0, 0)),
            pl.BlockSpec((1, D), lambda i: (0, 0)),
            pl.BlockSpec((D, D), lambda i: (0, 0)),
            pl.BlockSpec((1, D), lambda i: (0, 0)),
            pl.BlockSpec((1, D), lambda i: (0, 0)),
            pl.BlockSpec((1, D), lambda i: (0, 0)),
        ],
        out_specs=pl.BlockSpec((1, N_IMG_BLK, D), lambda i: (i, 0, 0)),
        out_shape=jax.ShapeDtypeStruct((B, N_IMG_BLK, D), jnp.float32),
        compiler_params=pltpu.CompilerParams(
            dimension_semantics=("arbitrary",),
        ),
    )(img_tokens, W1, b1.reshape(1, D), W2, b2.reshape(1, D),
      vstart.reshape(1, D), vend.reshape(1, D))


def kernel(img_tokens, W1, b1, W2, b2, vstart, vend, table,
           inst_ids, ans_ids, cu_inst, cu_ans):
    ni_pad = _round_up(inst_ids.shape[0], 16)
    na_pad = _round_up(ans_ids.shape[0], 16)
    inst_p = jnp.pad(inst_ids, (0, ni_pad - inst_ids.shape[0]))
    ans_p = jnp.pad(ans_ids, (0, na_pad - ans_ids.shape[0]))
    cu16 = jnp.zeros((16,), jnp.int32)
    cu16 = cu16.at[0:B + 1].set(cu_inst)
    cu16 = cu16.at[8:8 + B + 1].set(cu_ans)

    img_block = _mlp(img_tokens, W1, b1, W2, b2, vstart, vend)
    out_flat = _sc_gather(table, inst_p, ans_p, cu16,
                          img_block.reshape(B * N_IMG_BLK, D))
    return out_flat.reshape(B, MAXLEN, D)
